# baseline (speedup 1.0000x reference)
import functools

import jax
import jax.numpy as jnp
from jax.experimental import pallas as pl
from jax.experimental.pallas import tpu as pltpu

HIDDEN = 2048
NUM_EXPERTS = 16
TOP_K = 2


def _router_kernel(x_ref, wt_ref, logits_ref, probs_ref, weights_ref, idx_ref):
    x_blk = x_ref[...]
    wt = wt_ref[...]
    logits = jax.lax.dot_general(
        x_blk, wt, (((1,), (0,)), ((), ())),
        preferred_element_type=jnp.float32)
    logits_ref[...] = logits

    m = jnp.max(logits, axis=-1, keepdims=True)
    e = jnp.exp(logits - m)
    s = jnp.sum(e, axis=-1, keepdims=True)
    probs = e / s
    probs_ref[...] = probs

    iota = jax.lax.broadcasted_iota(jnp.int32, probs.shape, 1)
    p1 = jnp.max(probs, axis=-1, keepdims=True)
    i1 = jnp.argmax(probs, axis=-1, keepdims=True).astype(jnp.int32)
    masked = jnp.where(iota == i1, -jnp.inf, probs)
    p2 = jnp.max(masked, axis=-1, keepdims=True)
    i2 = jnp.argmax(masked, axis=-1, keepdims=True).astype(jnp.int32)
    denom = p1 + p2
    weights_ref[...] = jnp.concatenate([p1 / denom, p2 / denom], axis=-1)
    idx_ref[...] = jnp.concatenate([i1, i2], axis=-1)


@jax.jit
def kernel(x, W):
    B, S, H = x.shape
    N = B * S
    x2 = x.reshape(N, H)
    wt = W.T

    block_rows = 512
    grid = (N // block_rows,)

    logits, probs, weights, idx = pl.pallas_call(
        _router_kernel,
        grid=grid,
        compiler_params=pltpu.CompilerParams(
            dimension_semantics=("arbitrary",)),
        in_specs=[
            pl.BlockSpec((block_rows, H), lambda i: (i, 0)),
            pl.BlockSpec((H, NUM_EXPERTS), lambda i: (0, 0)),
        ],
        out_specs=[
            pl.BlockSpec((block_rows, NUM_EXPERTS), lambda i: (i, 0)),
            pl.BlockSpec((block_rows, NUM_EXPERTS), lambda i: (i, 0)),
            pl.BlockSpec((block_rows, TOP_K), lambda i: (i, 0)),
            pl.BlockSpec((block_rows, TOP_K), lambda i: (i, 0)),
        ],
        out_shape=[
            jax.ShapeDtypeStruct((N, NUM_EXPERTS), jnp.float32),
            jax.ShapeDtypeStruct((N, NUM_EXPERTS), jnp.float32),
            jax.ShapeDtypeStruct((N, TOP_K), jnp.float32),
            jax.ShapeDtypeStruct((N, TOP_K), jnp.int32),
        ],
    )(x2, wt)

    routing_weights = weights.reshape(B, S, TOP_K)
    expert_indices = idx.reshape(B, S, TOP_K)
    return (routing_weights, expert_indices, logits, probs)
